# Initial kernel scaffold; baseline (speedup 1.0000x reference)
import jax
import jax.numpy as jnp
from jax.experimental import pallas as pl


def kernel(x, emb_0, emb_1, emb_2, emb_3, emb_4, emb_5, emb_6, emb_7, emb_8, emb_9, emb_10, emb_11, emb_12, emb_13, emb_14, emb_15):
    raise NotImplementedError("write your pallas kernel here")


# speedup vs baseline: 107.5668x; 107.5668x over previous
import jax
import jax.numpy as jnp
import numpy as np
from jax import lax
from jax.experimental import pallas as pl
from jax.experimental.pallas import tpu as pltpu
from jax.experimental.pallas import tpu_sc as plsc

_N_LEVELS = 16
_NF = 2
_LOG2_T = 19
_T = 2 ** _LOG2_T
_BASE_RES = 16
_FINEST_RES = 1024
_B_PTS = 1048576
_GROWTH = np.float32(
    np.exp((np.log(np.float32(_FINEST_RES)) - np.log(np.float32(_BASE_RES)))
           / (_N_LEVELS - 1)))
_PRIME1_I32 = int(np.uint32(2654435761).view(np.int32))
_HASH_MASK = _T - 1

_RES = []
_GS = []
_TSIZE = []
_DENSE = []
for _i in range(_N_LEVELS):
    _resf = float(np.floor(np.float32(_BASE_RES) * _GROWTH ** np.float32(_i)))
    _r = int(_resf)
    _RES.append(_r)
    _GS.append(np.float32(1.0 / _resf))
    if _r * _r < _T:
        _TSIZE.append((_r + 1) ** 2)
        _DENSE.append(True)
    else:
        _TSIZE.append(_T)
        _DENSE.append(False)

_NC = 2
_NS = 16
_NW = _NC * _NS
_PW = _B_PTS // _NW
_C = 128
_VPC = _C // 16
_NCHUNK = _PW // _C

_RES_LEVELS = list(range(0, 9))
_BIG_LEVELS = list(range(9, 16))
_NGATH = [2 if _DENSE[l] else 4 for l in _BIG_LEVELS]
_GW = 8
_GOFF = list(np.cumsum([0] + _NGATH))
_TOTG = _GOFF[-1]


def _coords(x0, x1, l):
    gs = _GS[l]
    b0 = (x0 / gs).astype(jnp.int32)
    b1 = (x1 / gs).astype(jnp.int32)
    return b0, b1


def _weights(x0, x1, b0, b1, l):
    gs = _GS[l]
    gmin0 = b0.astype(jnp.float32) * gs
    gmin1 = b1.astype(jnp.float32) * gs
    w0 = (x0 - gmin0) / ((gmin0 + gs) - gmin0)
    w1 = (x1 - gmin1) / ((gmin1 + gs) - gmin1)
    return w0, w1


def _corner_indices(b0, b1, l):
    if _DENSE[l]:
        r = _RES[l]
        i00 = b0 * r + b1
        return i00, i00 + 1, i00 + r, i00 + r + 1
    h0 = b0 ^ (b1 * _PRIME1_I32)
    h1 = b0 ^ ((b1 + 1) * _PRIME1_I32)
    h2 = (b0 + 1) ^ (b1 * _PRIME1_I32)
    h3 = (b0 + 1) ^ ((b1 + 1) * _PRIME1_I32)
    return (h0 & _HASH_MASK, h1 & _HASH_MASK,
            h2 & _HASH_MASK, h3 & _HASH_MASK)


def _lerp_store(corner_vals, w0, w1, out_ref, prow, l):
    u0 = 1.0 - w0
    u1 = 1.0 - w1
    for f in range(_NF):
        e00, e01, e10, e11 = corner_vals[f]
        c0 = e00 * u1 + e01 * w1
        c1 = e10 * u1 + e11 * w1
        o = c0 * u0 + c1 * w0
        plsc.store_scatter(
            out_ref, [prow, jnp.full((16,), 2 * l + f, jnp.int32)], o)


def _sc_body(x0_hbm, x1_hbm, *refs):
    tbl_hbm = refs[0:_N_LEVELS]
    out_hbm = refs[_N_LEVELS]
    s = refs[_N_LEVELS + 1:]
    nres = len(_RES_LEVELS)
    tbl_v = s[0:nres]
    x0_v, x1_v, out_v = s[nres], s[nres + 1], s[nres + 2]
    idx_v = s[nres + 3:nres + 3 + _TOTG]
    rows_v = s[nres + 3 + _TOTG:nres + 3 + 2 * _TOTG]
    gsem = s[nres + 3 + 2 * _TOTG]

    wid = lax.axis_index("s") * _NC + lax.axis_index("c")
    base = wid * _PW

    for li, l in enumerate(_RES_LEVELS):
        pltpu.sync_copy(tbl_hbm[l], tbl_v[li])

    lane = lax.iota(jnp.int32, 16)

    def chunk_body(t, carry):
        off = base + t * _C
        pltpu.sync_copy(x0_hbm.at[pl.ds(off, _C)], x0_v)
        pltpu.sync_copy(x1_hbm.at[pl.ds(off, _C)], x1_v)

        def idx_body(v, c):
            x0 = x0_v[pl.ds(v * 16, 16)]
            x1 = x1_v[pl.ds(v * 16, 16)]
            for li, l in enumerate(_BIG_LEVELS):
                b0, b1 = _coords(x0, x1, l)
                i00, i01, i10, i11 = _corner_indices(b0, b1, l)
                if _DENSE[l]:
                    ivs = (i00, i10)
                else:
                    ivs = (i00, i01, i10, i11)
                for ci, iv in enumerate(ivs):
                    idx_v[_GOFF[li] + ci][pl.ds(v * 16, 16)] = iv
            return c

        lax.fori_loop(0, _VPC, idx_body, 0)

        copies = []
        for li, l in enumerate(_BIG_LEVELS):
            for ci in range(_NGATH[li]):
                g = _GOFF[li] + ci
                copies.append(pltpu.async_copy(
                    tbl_hbm[l].at[idx_v[g]], rows_v[g], gsem))

        def res_body(v, c):
            x0 = x0_v[pl.ds(v * 16, 16)]
            x1 = x1_v[pl.ds(v * 16, 16)]
            prow = v * 16 + lane
            for li, l in enumerate(_RES_LEVELS):
                b0, b1 = _coords(x0, x1, l)
                w0, w1 = _weights(x0, x1, b0, b1, l)
                i00, i01, i10, i11 = _corner_indices(b0, b1, l)
                vals = []
                for f in range(_NF):
                    vals.append(tuple(
                        plsc.load_gather(tbl_v[li], [2 * i + f])
                        for i in (i00, i01, i10, i11)))
                _lerp_store(vals, w0, w1, out_v, prow, l)
            return c

        lax.fori_loop(0, _VPC, res_body, 0)

        for cp in copies:
            cp.wait()

        def big_body(v, c):
            x0 = x0_v[pl.ds(v * 16, 16)]
            x1 = x1_v[pl.ds(v * 16, 16)]
            prow = v * 16 + lane
            lidx = prow
            for li, l in enumerate(_BIG_LEVELS):
                b0, b1 = _coords(x0, x1, l)
                w0, w1 = _weights(x0, x1, b0, b1, l)
                g = _GOFF[li]
                vals = []
                for f in range(_NF):
                    if _DENSE[l]:
                        col = jnp.full((16,), f, jnp.int32)
                        col2 = jnp.full((16,), 2 + f, jnp.int32)
                        e00 = plsc.load_gather(rows_v[g + 0], [lidx, col])
                        e01 = plsc.load_gather(rows_v[g + 0], [lidx, col2])
                        e10 = plsc.load_gather(rows_v[g + 1], [lidx, col])
                        e11 = plsc.load_gather(rows_v[g + 1], [lidx, col2])
                    else:
                        col = jnp.full((16,), f, jnp.int32)
                        e00 = plsc.load_gather(rows_v[g + 0], [lidx, col])
                        e01 = plsc.load_gather(rows_v[g + 1], [lidx, col])
                        e10 = plsc.load_gather(rows_v[g + 2], [lidx, col])
                        e11 = plsc.load_gather(rows_v[g + 3], [lidx, col])
                    vals.append((e00, e01, e10, e11))
                _lerp_store(vals, w0, w1, out_v, prow, l)
            return c

        lax.fori_loop(0, _VPC, big_body, 0)

        pltpu.sync_copy(out_v, out_hbm.at[pl.ds(off, _C)])
        return carry

    lax.fori_loop(0, _NCHUNK, chunk_body, 0)


def _make_kernel():
    scratch = []
    scratch += [pltpu.VMEM((_TSIZE[l] * _NF,), jnp.float32)
                for l in _RES_LEVELS]
    scratch += [pltpu.VMEM((_C,), jnp.float32),
                pltpu.VMEM((_C,), jnp.float32),
                pltpu.VMEM((_C, 2 * _N_LEVELS), jnp.float32)]
    scratch += [pltpu.VMEM((_C,), jnp.int32) for _ in range(_TOTG)]
    for li, l in enumerate(_BIG_LEVELS):
        scratch += [pltpu.VMEM((_C, _GW), jnp.float32)
                    for _ in range(_NGATH[li])]
    scratch += [pltpu.SemaphoreType.DMA]
    mesh = plsc.VectorSubcoreMesh(core_axis_name="c", subcore_axis_name="s")
    return pl.kernel(
        _sc_body,
        out_type=jax.ShapeDtypeStruct((_B_PTS, 2 * _N_LEVELS), jnp.float32),
        mesh=mesh,
        scratch_types=scratch,
        compiler_params=pltpu.CompilerParams(
            needs_layout_passes=False, use_tc_tiling_on_sc=False),
    )


_sc_kernel = _make_kernel()


@jax.jit
def kernel(x, emb_0, emb_1, emb_2, emb_3, emb_4, emb_5, emb_6, emb_7,
           emb_8, emb_9, emb_10, emb_11, emb_12, emb_13, emb_14, emb_15):
    tables = [emb_0, emb_1, emb_2, emb_3, emb_4, emb_5, emb_6, emb_7,
              emb_8, emb_9, emb_10, emb_11, emb_12, emb_13, emb_14, emb_15]
    args = []
    for l in range(_N_LEVELS):
        tb = tables[l]
        if l in _RES_LEVELS:
            args.append(tb.reshape(-1))
        elif _DENSE[l]:
            pad = jnp.zeros((tb.shape[0], _GW - 2 * _NF), tb.dtype)
            args.append(jnp.concatenate(
                [tb, jnp.roll(tb, -1, axis=0), pad], axis=1))
        else:
            pad = jnp.zeros((tb.shape[0], _GW - _NF), tb.dtype)
            args.append(jnp.concatenate([tb, pad], axis=1))
    xt = x.T
    return _sc_kernel(xt[0], xt[1], *args)
